# Initial kernel scaffold; baseline (speedup 1.0000x reference)
import jax
import jax.numpy as jnp
from jax.experimental import pallas as pl


def kernel(x1, x2, W1, b1, W2, b2, W3, b3, V1, c1, V2, c2):
    raise NotImplementedError("write your pallas kernel here")


# speedup vs baseline: 1.2618x; 1.2618x over previous
import functools
import math

import jax
import jax.numpy as jnp
from jax.experimental import pallas as pl

D_X = 4096
H = 1024
NL = 64

_LOG_2PI = math.log(2.0 * math.pi)


def _fused_kernel(x_ref, w1_ref, b1_ref, w2_ref, b2_ref, w3_ref, b3_ref,
                  v1_ref, c1_ref, v2_ref, c2_ref,
                  g_ref, p1_ref, p2_ref, z1_ref, z2_ref,
                  e1_ref, e2_ref, inter_ref, logq_ref):
    x = x_ref[...]
    h = jax.nn.relu(jnp.dot(x, w1_ref[...],
                            preferred_element_type=jnp.float32) + b1_ref[...])
    h = jax.nn.relu(jnp.dot(h, w2_ref[...],
                            preferred_element_type=jnp.float32) + b2_ref[...])
    o = jnp.dot(h, w3_ref[...], preferred_element_type=jnp.float32) + b3_ref[...]

    e1_mean = o[0:1, 0:NL]
    e1_logstd = o[0:1, NL:2 * NL]
    e2_mean = o[1:2, 0:NL]
    e2_logstd = o[1:2, NL:2 * NL]
    e1_std = jnp.exp(e1_logstd)
    e2_std = jnp.exp(e2_logstd)

    d = jnp.abs(e1_mean - e2_mean)
    hh = jax.nn.relu(jnp.dot(d, v1_ref[...],
                             preferred_element_type=jnp.float32) + c1_ref[...])
    logits = jnp.dot(hh, v2_ref[...],
                     preferred_element_type=jnp.float32) + c2_ref[...]
    logp = jax.nn.log_softmax(logits, axis=-1)

    score = logp + g_ref[...]
    iota65 = jax.lax.broadcasted_iota(jnp.int32, (1, NL + 1), 1)
    smax = jnp.max(score)
    idx = jnp.min(jnp.where(score >= smax, iota65, NL + 1))

    onehot = (iota65 == idx).astype(jnp.float32)
    log_q_I = jnp.sum(onehot * logp)

    iota64 = jax.lax.broadcasted_iota(jnp.int32, (1, NL), 1)
    i_mask = iota64 == (idx - 1)

    p1 = p1_ref[...]
    p2 = p2_ref[...]
    eps_mean = jnp.where(i_mask, e1_mean, p1 * e1_mean + (1.0 - p1) * e2_mean)
    eps_std = jnp.where(i_mask, e1_std, p2 * e1_std + (1.0 - p2) * e2_std)

    e1 = eps_mean + jnp.sqrt(eps_std) * z1_ref[...]
    log_q_e1 = -0.5 * jnp.sum((e1 - eps_mean) ** 2 / eps_std
                              + jnp.log(eps_std) + _LOG_2PI)

    e2_samp = e2_mean + jnp.sqrt(e2_std) * z2_ref[...]
    e2 = jnp.where(i_mask, e2_samp, e1)
    per_dim = -0.5 * ((e2 - e2_mean) ** 2 / e2_std + jnp.log(e2_std) + _LOG_2PI)
    log_q_e2 = jnp.sum(jnp.where(i_mask, per_dim, 0.0))

    e1_ref[...] = e1
    e2_ref[...] = e2
    inter_ref[...] = onehot
    logq_ref[...] = jnp.full((1, 1), log_q_e1 + log_q_e2 + log_q_I,
                             dtype=jnp.float32)


@functools.partial(jax.jit, static_argnames=("interpret",))
def _run(x1, x2, W1, b1, W2, b2, W3, b3, V1, c1, V2, c2, interpret=False):
    skey = jax.random.key(1234)
    g = jax.random.gumbel(jax.random.fold_in(skey, 0), (NL + 1,), jnp.float32)
    p1 = jax.random.uniform(jax.random.fold_in(skey, 1), (NL,), jnp.float32)
    p2 = jax.random.uniform(jax.random.fold_in(skey, 2), (NL,), jnp.float32)
    z1 = jax.random.normal(jax.random.fold_in(skey, 3), (NL,), jnp.float32)
    z2 = jax.random.normal(jax.random.fold_in(skey, 4), (NL,), jnp.float32)

    X = jnp.zeros((8, D_X), jnp.float32).at[0].set(x1).at[1].set(x2)

    out_shapes = (
        jax.ShapeDtypeStruct((1, NL), jnp.float32),
        jax.ShapeDtypeStruct((1, NL), jnp.float32),
        jax.ShapeDtypeStruct((1, NL + 1), jnp.float32),
        jax.ShapeDtypeStruct((1, 1), jnp.float32),
    )
    e1, e2, inter, logq = pl.pallas_call(
        _fused_kernel,
        out_shape=out_shapes,
        interpret=interpret,
    )(X, W1, b1.reshape(1, H), W2, b2.reshape(1, H), W3,
      b3.reshape(1, 2 * NL), V1, c1.reshape(1, 256), V2,
      c2.reshape(1, NL + 1), g.reshape(1, NL + 1), p1.reshape(1, NL),
      p2.reshape(1, NL), z1.reshape(1, NL), z2.reshape(1, NL))
    return ((e1.reshape(NL), e2.reshape(NL), inter.reshape(NL + 1)),
            logq.reshape(()))


def kernel(x1, x2, W1, b1, W2, b2, W3, b3, V1, c1, V2, c2):
    return _run(x1, x2, W1, b1, W2, b2, W3, b3, V1, c1, V2, c2)
